# speedup vs baseline: 4.0593x; 4.0593x over previous
import functools

import jax
import jax.numpy as jnp
from jax import lax
from jax.experimental import pallas as pl
from jax.experimental.pallas import tpu as pltpu
from jax.experimental.pallas import tpu_sc as plsc

N = 10000
E = 320000
F = 128
NG = 64

NC = 2
NS = 16
NW = NC * NS
L = 16

E_PER_W = E // NW


def _deg_kernel_body(dst_hbm, deg_hbm, idx_v, ones_v, zeros_v, deg_sh):
    cid = lax.axis_index("c")
    sid = lax.axis_index("s")
    wid = cid * NS + sid
    base = wid * E_PER_W

    def fill(i, _):
        ones_v[pl.ds(i * L, L)] = jnp.full((L,), 1.0, jnp.float32)
        return ()
    lax.fori_loop(0, E_PER_W // L, fill, ())

    @pl.when(sid == 0)
    def _():
        def zfill(i, _):
            zeros_v[pl.ds(i * L, L)] = jnp.zeros((L,), jnp.float32)
            return ()
        lax.fori_loop(0, N // L, zfill, ())
        pltpu.sync_copy(zeros_v, deg_sh)

    plsc.subcore_barrier()

    pltpu.sync_copy(dst_hbm.at[pl.ds(base, E_PER_W)], idx_v)
    pltpu.sync_copy(ones_v, deg_sh.at[idx_v], add=True)

    plsc.subcore_barrier()

    @pl.when(sid == 0)
    def _():
        pltpu.sync_copy(deg_sh, deg_hbm.at[cid])


def _deg_partials(dst):
    mesh = plsc.VectorSubcoreMesh(core_axis_name="c", subcore_axis_name="s")
    return pl.kernel(
        _deg_kernel_body,
        out_type=jax.ShapeDtypeStruct((NC, N), jnp.float32),
        mesh=mesh,
        scratch_types=[
            pltpu.VMEM((E_PER_W,), jnp.int32),
            pltpu.VMEM((E_PER_W,), jnp.float32),
            pltpu.VMEM((N,), jnp.float32),
            pltpu.VMEM_SHARED((N,), jnp.float32),
        ],
    )(dst)


def kernel(x, edge_index, batch, Wc, bc, W1, b1, W2, b2, W3, b3):
    src = edge_index[0]
    dst = edge_index[1]

    degp = _deg_partials(dst)
    deg = 1.0 + degp[0] + degp[1]
    dinv = lax.rsqrt(deg)

    xw = x @ Wc
    y = xw * dinv[:, None]
    acc = jnp.zeros((N, F), jnp.float32).at[dst].add(y[src])
    h = jax.nn.relu(dinv[:, None] * (acc + y) + bc)

    gmax = jax.ops.segment_max(h, batch, num_segments=NG)
    s = jax.ops.segment_sum(h, batch, num_segments=NG)
    cnt = jax.ops.segment_sum(jnp.ones((N,), jnp.float32), batch, num_segments=NG)
    gmean = s / jnp.maximum(cnt, 1.0)[:, None]
    out = jax.nn.relu(jnp.concatenate([gmax, gmean], axis=1))
    out = jax.nn.relu(out @ W1 + b1)
    out = jax.nn.relu(out @ W2 + b2)
    out = out @ W3 + b3
    return jax.nn.log_softmax(out, axis=-1)


# speedup vs baseline: 19.2693x; 4.7469x over previous
import functools

import jax
import jax.numpy as jnp
from jax import lax
from jax.experimental import pallas as pl
from jax.experimental.pallas import tpu as pltpu
from jax.experimental.pallas import tpu_sc as plsc

N = 10000
E = 320000
F = 128
NG = 64

NC = 2
NS = 16
NW = NC * NS
L = 16

E_PER_W = E // NW


def _deg_kernel_body(dst_hbm, deg_hbm, idx_v, ones_v, zeros_v, deg_sh):
    cid = lax.axis_index("c")
    sid = lax.axis_index("s")
    wid = cid * NS + sid
    base = wid * E_PER_W

    def fill(i, _):
        ones_v[pl.ds(i * L, L)] = jnp.full((L,), 1.0, jnp.float32)
        return ()
    lax.fori_loop(0, E_PER_W // L, fill, ())

    @pl.when(sid == 0)
    def _():
        def zfill(i, _):
            zeros_v[pl.ds(i * L, L)] = jnp.zeros((L,), jnp.float32)
            return ()
        lax.fori_loop(0, N // L, zfill, ())
        pltpu.sync_copy(zeros_v, deg_sh)

    plsc.subcore_barrier()

    pltpu.sync_copy(dst_hbm.at[pl.ds(base, E_PER_W)], idx_v)
    pltpu.sync_copy(ones_v, deg_sh.at[idx_v], add=True)

    plsc.subcore_barrier()

    @pl.when(sid == 0)
    def _():
        pltpu.sync_copy(deg_sh, deg_hbm.at[cid])


def _deg_partials(dst):
    mesh = plsc.VectorSubcoreMesh(core_axis_name="c", subcore_axis_name="s")
    return pl.kernel(
        _deg_kernel_body,
        out_type=jax.ShapeDtypeStruct((NC, N), jnp.float32),
        mesh=mesh,
        scratch_types=[
            pltpu.VMEM((E_PER_W,), jnp.int32),
            pltpu.VMEM((E_PER_W,), jnp.float32),
            pltpu.VMEM((N,), jnp.float32),
            pltpu.VMEM_SHARED((N,), jnp.float32),
        ],
    )(dst)


K = 125
NCHUNKS = E_PER_W // K
ROW_STRIDE = 624
ROW_SPAN = 640


def _agg_body(y_hbm, srcw_hbm, dstw_hbm, zeros_hbm, acc_hbm,
              src_v, dst_v, rows_v, sem, acc_sh):
    cid = lax.axis_index("c")
    sid = lax.axis_index("s")
    wid = cid * NS + sid

    rbase = sid * ROW_STRIDE
    pltpu.sync_copy(zeros_hbm.at[pl.ds(rbase, ROW_SPAN)],
                    acc_sh.at[pl.ds(rbase, ROW_SPAN)])

    pltpu.sync_copy(srcw_hbm.at[wid], src_v)
    pltpu.sync_copy(dstw_hbm.at[wid], dst_v)

    plsc.subcore_barrier()

    def chunk(j, _):
        pltpu.async_copy(y_hbm.at[src_v.at[j, 0]], rows_v, sem).wait()
        pltpu.sync_copy(rows_v, acc_sh.at[dst_v.at[j, 0]], add=True)
        return ()
    lax.fori_loop(0, NCHUNKS, chunk, ())

    plsc.subcore_barrier()

    pltpu.sync_copy(acc_sh.at[pl.ds(rbase, ROW_SPAN)],
                    acc_hbm.at[cid, pl.ds(rbase, ROW_SPAN)])


def _agg_partials(y, src, dst):
    srcw = src.reshape(NW, NCHUNKS, 1, K)
    dstw = dst.reshape(NW, NCHUNKS, 1, K)
    zeros = jnp.zeros((N, F), jnp.float32)
    mesh = plsc.VectorSubcoreMesh(core_axis_name="c", subcore_axis_name="s")
    return pl.kernel(
        _agg_body,
        out_type=jax.ShapeDtypeStruct((NC, N, F), jnp.float32),
        mesh=mesh,
        scratch_types=[
            pltpu.VMEM((NCHUNKS, 1, K), jnp.int32),
            pltpu.VMEM((NCHUNKS, 1, K), jnp.int32),
            pltpu.VMEM((K, F), jnp.float32),
            pltpu.SemaphoreType.DMA,
            pltpu.VMEM_SHARED((N, F), jnp.float32),
        ],
    )(y, srcw, dstw, zeros)


def kernel(x, edge_index, batch, Wc, bc, W1, b1, W2, b2, W3, b3):
    src = edge_index[0]
    dst = edge_index[1]

    degp = _deg_partials(dst)
    deg = 1.0 + degp[0] + degp[1]
    dinv = lax.rsqrt(deg)

    xw = x @ Wc
    y = xw * dinv[:, None]
    accp = _agg_partials(y, src, dst)
    acc = accp[0] + accp[1]
    h = jax.nn.relu(dinv[:, None] * (acc + y) + bc)

    gmax = jax.ops.segment_max(h, batch, num_segments=NG)
    s = jax.ops.segment_sum(h, batch, num_segments=NG)
    cnt = jax.ops.segment_sum(jnp.ones((N,), jnp.float32), batch, num_segments=NG)
    gmean = s / jnp.maximum(cnt, 1.0)[:, None]
    out = jax.nn.relu(jnp.concatenate([gmax, gmean], axis=1))
    out = jax.nn.relu(out @ W1 + b1)
    out = jax.nn.relu(out @ W2 + b2)
    out = out @ W3 + b3
    return jax.nn.log_softmax(out, axis=-1)


# speedup vs baseline: 20.1374x; 1.0451x over previous
import functools

import jax
import jax.numpy as jnp
from jax import lax
from jax.experimental import pallas as pl
from jax.experimental.pallas import tpu as pltpu
from jax.experimental.pallas import tpu_sc as plsc

N = 10000
E = 320000
F = 128
NG = 64

NC = 2
NS = 16
NW = NC * NS
L = 16

E_PER_W = E // NW


def _deg_kernel_body(dst_hbm, deg_hbm, idx_v, ones_v, zeros_v, deg_sh):
    cid = lax.axis_index("c")
    sid = lax.axis_index("s")
    wid = cid * NS + sid
    base = wid * E_PER_W

    def fill(i, _):
        ones_v[pl.ds(i * L, L)] = jnp.full((L,), 1.0, jnp.float32)
        return ()
    lax.fori_loop(0, E_PER_W // L, fill, ())

    @pl.when(sid == 0)
    def _():
        def zfill(i, _):
            zeros_v[pl.ds(i * L, L)] = jnp.zeros((L,), jnp.float32)
            return ()
        lax.fori_loop(0, N // L, zfill, ())
        pltpu.sync_copy(zeros_v, deg_sh)

    plsc.subcore_barrier()

    pltpu.sync_copy(dst_hbm.at[pl.ds(base, E_PER_W)], idx_v)
    pltpu.sync_copy(ones_v, deg_sh.at[idx_v], add=True)

    plsc.subcore_barrier()

    @pl.when(sid == 0)
    def _():
        pltpu.sync_copy(deg_sh, deg_hbm.at[cid])


def _deg_partials(dst):
    mesh = plsc.VectorSubcoreMesh(core_axis_name="c", subcore_axis_name="s")
    return pl.kernel(
        _deg_kernel_body,
        out_type=jax.ShapeDtypeStruct((NC, N), jnp.float32),
        mesh=mesh,
        scratch_types=[
            pltpu.VMEM((E_PER_W,), jnp.int32),
            pltpu.VMEM((E_PER_W,), jnp.float32),
            pltpu.VMEM((N,), jnp.float32),
            pltpu.VMEM_SHARED((N,), jnp.float32),
        ],
    )(dst)


K = 40
NCHUNKS = E_PER_W // K
ROW_STRIDE = 624
ROW_SPAN = 640


def _agg_body(y_hbm, srcw_hbm, dstw_hbm, zeros_hbm, acc_hbm,
              src_v, dst_v, rows_v, rows_b, sem, sem_b, acc_sh):
    cid = lax.axis_index("c")
    sid = lax.axis_index("s")
    wid = cid * NS + sid

    rbase = sid * ROW_STRIDE
    pltpu.sync_copy(zeros_hbm.at[pl.ds(rbase, ROW_SPAN)],
                    acc_sh.at[pl.ds(rbase, ROW_SPAN)])

    ebase = wid * E_PER_W
    pltpu.sync_copy(srcw_hbm.at[pl.ds(ebase, E_PER_W)], src_v)
    pltpu.sync_copy(dstw_hbm.at[pl.ds(ebase, E_PER_W)], dst_v)

    plsc.subcore_barrier()

    def gather(j, buf, sm):
        return pltpu.async_copy(y_hbm.at[src_v.at[pl.ds(j * K, K)]], buf, sm)

    def gwait(j, buf, sm):
        pltpu.make_async_copy(y_hbm.at[src_v.at[pl.ds(j * K, K)]], buf, sm).wait()

    def scat(j, buf):
        pltpu.sync_copy(buf, acc_sh.at[dst_v.at[pl.ds(j * K, K)]], add=True)

    gather(0, rows_v, sem)

    def pair(i, _):
        j = 2 * i
        gather(j + 1, rows_b, sem_b)
        gwait(j, rows_v, sem)
        scat(j, rows_v)
        gather(j + 2, rows_v, sem)
        gwait(j + 1, rows_b, sem_b)
        scat(j + 1, rows_b)
        return ()
    lax.fori_loop(0, NCHUNKS // 2 - 1, pair, ())

    jlast = NCHUNKS - 2
    gather(jlast + 1, rows_b, sem_b)
    gwait(jlast, rows_v, sem)
    scat(jlast, rows_v)
    gwait(jlast + 1, rows_b, sem_b)
    scat(jlast + 1, rows_b)

    plsc.subcore_barrier()

    pltpu.sync_copy(acc_sh.at[pl.ds(rbase, ROW_SPAN)],
                    acc_hbm.at[cid, pl.ds(rbase, ROW_SPAN)])


def _agg_partials(y, src, dst):
    zeros = jnp.zeros((N, F), jnp.float32)
    mesh = plsc.VectorSubcoreMesh(core_axis_name="c", subcore_axis_name="s")
    return pl.kernel(
        _agg_body,
        out_type=jax.ShapeDtypeStruct((NC, N, F), jnp.float32),
        mesh=mesh,
        scratch_types=[
            pltpu.VMEM((E_PER_W,), jnp.int32),
            pltpu.VMEM((E_PER_W,), jnp.int32),
            pltpu.VMEM((K, F), jnp.float32),
            pltpu.VMEM((K, F), jnp.float32),
            pltpu.SemaphoreType.DMA,
            pltpu.SemaphoreType.DMA,
            pltpu.VMEM_SHARED((N, F), jnp.float32),
        ],
    )(y, src, dst, zeros)


def kernel(x, edge_index, batch, Wc, bc, W1, b1, W2, b2, W3, b3):
    src = edge_index[0]
    dst = edge_index[1]

    degp = _deg_partials(dst)
    deg = 1.0 + degp[0] + degp[1]
    dinv = lax.rsqrt(deg)

    xw = x @ Wc
    y = xw * dinv[:, None]
    accp = _agg_partials(y, src, dst)
    acc = accp[0] + accp[1]
    h = jax.nn.relu(dinv[:, None] * (acc + y) + bc)

    gmax = jax.ops.segment_max(h, batch, num_segments=NG)
    s = jax.ops.segment_sum(h, batch, num_segments=NG)
    cnt = jax.ops.segment_sum(jnp.ones((N,), jnp.float32), batch, num_segments=NG)
    gmean = s / jnp.maximum(cnt, 1.0)[:, None]
    out = jax.nn.relu(jnp.concatenate([gmax, gmean], axis=1))
    out = jax.nn.relu(out @ W1 + b1)
    out = jax.nn.relu(out @ W2 + b2)
    out = out @ W3 + b3
    return jax.nn.log_softmax(out, axis=-1)


# speedup vs baseline: 28.9592x; 1.4381x over previous
import functools

import jax
import jax.numpy as jnp
from jax import lax
from jax.experimental import pallas as pl
from jax.experimental.pallas import tpu as pltpu
from jax.experimental.pallas import tpu_sc as plsc

N = 10000
E = 320000
F = 128
NG = 64

NC = 2
NS = 16
NW = NC * NS
L = 16

E_PER_W = E // NW


def _deg_kernel_body(dst_hbm, deg_hbm, idx_v, ones_v, zeros_v, deg_sh):
    cid = lax.axis_index("c")
    sid = lax.axis_index("s")
    wid = cid * NS + sid
    base = wid * E_PER_W

    def fill(i, _):
        ones_v[pl.ds(i * L, L)] = jnp.full((L,), 1.0, jnp.float32)
        return ()
    lax.fori_loop(0, E_PER_W // L, fill, ())

    @pl.when(sid == 0)
    def _():
        def zfill(i, _):
            zeros_v[pl.ds(i * L, L)] = jnp.zeros((L,), jnp.float32)
            return ()
        lax.fori_loop(0, N // L, zfill, ())
        pltpu.sync_copy(zeros_v, deg_sh)

    plsc.subcore_barrier()

    pltpu.sync_copy(dst_hbm.at[pl.ds(base, E_PER_W)], idx_v)
    pltpu.sync_copy(ones_v, deg_sh.at[idx_v], add=True)

    plsc.subcore_barrier()

    @pl.when(sid == 0)
    def _():
        pltpu.sync_copy(deg_sh, deg_hbm.at[cid])


def _deg_partials(dst):
    mesh = plsc.VectorSubcoreMesh(core_axis_name="c", subcore_axis_name="s")
    return pl.kernel(
        _deg_kernel_body,
        out_type=jax.ShapeDtypeStruct((NC, N), jnp.float32),
        mesh=mesh,
        scratch_types=[
            pltpu.VMEM((E_PER_W,), jnp.int32),
            pltpu.VMEM((E_PER_W,), jnp.float32),
            pltpu.VMEM((N,), jnp.float32),
            pltpu.VMEM_SHARED((N,), jnp.float32),
        ],
    )(dst)


K = 40
NCHUNKS = E_PER_W // K
ROW_STRIDE = 624
ROW_SPAN = 640


def _agg_body(y_hbm, srcw_hbm, dstw_hbm, zeros_hbm, acc_hbm,
              src_v, dst_v, rows_v, rows_b, sem, sem_b, acc_sh):
    cid = lax.axis_index("c")
    sid = lax.axis_index("s")
    wid = cid * NS + sid

    rbase = sid * ROW_STRIDE
    pltpu.sync_copy(zeros_hbm.at[pl.ds(rbase, ROW_SPAN)],
                    acc_sh.at[pl.ds(rbase, ROW_SPAN)])

    ebase = wid * E_PER_W
    pltpu.sync_copy(srcw_hbm.at[pl.ds(ebase, E_PER_W)], src_v)
    pltpu.sync_copy(dstw_hbm.at[pl.ds(ebase, E_PER_W)], dst_v)

    plsc.subcore_barrier()

    def gather(j, buf, sm):
        return pltpu.async_copy(y_hbm.at[src_v.at[pl.ds(j * K, K)]], buf, sm)

    def gwait(j, buf, sm):
        pltpu.make_async_copy(y_hbm.at[src_v.at[pl.ds(j * K, K)]], buf, sm).wait()

    def scat(j, buf):
        pltpu.sync_copy(buf, acc_sh.at[dst_v.at[pl.ds(j * K, K)]], add=True)

    gather(0, rows_v, sem)

    def pair(i, _):
        j = 2 * i
        gather(j + 1, rows_b, sem_b)
        gwait(j, rows_v, sem)
        scat(j, rows_v)
        gather(j + 2, rows_v, sem)
        gwait(j + 1, rows_b, sem_b)
        scat(j + 1, rows_b)
        return ()
    lax.fori_loop(0, NCHUNKS // 2 - 1, pair, ())

    jlast = NCHUNKS - 2
    gather(jlast + 1, rows_b, sem_b)
    gwait(jlast, rows_v, sem)
    scat(jlast, rows_v)
    gwait(jlast + 1, rows_b, sem_b)
    scat(jlast + 1, rows_b)

    plsc.subcore_barrier()

    pltpu.sync_copy(acc_sh.at[pl.ds(rbase, ROW_SPAN)],
                    acc_hbm.at[cid, pl.ds(rbase, ROW_SPAN)])


def _agg_partials(y, src, dst):
    zeros = jnp.zeros((N, F), jnp.float32)
    mesh = plsc.VectorSubcoreMesh(core_axis_name="c", subcore_axis_name="s")
    return pl.kernel(
        _agg_body,
        out_type=jax.ShapeDtypeStruct((NC, N, F), jnp.float32),
        mesh=mesh,
        scratch_types=[
            pltpu.VMEM((E_PER_W,), jnp.int32),
            pltpu.VMEM((E_PER_W,), jnp.int32),
            pltpu.VMEM((K, F), jnp.float32),
            pltpu.VMEM((K, F), jnp.float32),
            pltpu.SemaphoreType.DMA,
            pltpu.SemaphoreType.DMA,
            pltpu.VMEM_SHARED((N, F), jnp.float32),
        ],
    )(y, src, dst, zeros)


RC = 80
NRC = N // RC
_GDN = lax.GatherDimensionNumbers(
    offset_dims=(), collapsed_slice_dims=(0,), start_index_map=(0,))


def _lane_bcast(vec, lane_vec):
    return lax.gather(vec, lane_vec[:, None], _GDN, slice_sizes=(1,),
                      mode=lax.GatherScatterMode.PROMISE_IN_BOUNDS)


def _pool_body(acc0f, acc1f, yf, dinv_hbm, batch_hbm, bc_hbm,
               pmax_hbm, psum_hbm,
               a0_v, a1_v, y_v, dv_v, bt_v, bc_v, pm_v, ps_v):
    cid = lax.axis_index("c")
    sid = lax.axis_index("s")
    wid = cid * NS + sid

    pltpu.sync_copy(bc_hbm, bc_v)

    def pinit(i, _):
        pm_v[pl.ds(i * L, L)] = jnp.full((L,), -jnp.inf, jnp.float32)
        ps_v[pl.ds(i * L, L)] = jnp.zeros((L,), jnp.float32)
        return ()
    lax.fori_loop(0, (NG * F) // L, pinit, ())

    def do_chunk(i, _):
        c = wid + NW * i

        @pl.when(c < NRC)
        def _():
            base = c * RC
            pltpu.sync_copy(acc0f.at[pl.ds(base * F, RC * F)], a0_v)
            pltpu.sync_copy(acc1f.at[pl.ds(base * F, RC * F)], a1_v)
            pltpu.sync_copy(yf.at[pl.ds(base * F, RC * F)], y_v)
            pltpu.sync_copy(dinv_hbm.at[pl.ds(base, RC)], dv_v.at[pl.ds(0, RC)])
            pltpu.sync_copy(batch_hbm.at[pl.ds(base, RC)], bt_v.at[pl.ds(0, RC)])

            def row(r, _):
                dvr = jnp.full((L,), dv_v[pl.ds(r, L)][0], jnp.float32)
                ioff = bt_v[pl.ds(r, L)][0] * F + lax.iota(jnp.int32, L)
                for g in range(F // L):
                    o = r * F + g * L
                    t = a0_v[pl.ds(o, L)] + a1_v[pl.ds(o, L)] + y_v[pl.ds(o, L)]
                    h = jnp.maximum(dvr * t + bc_v[pl.ds(g * L, L)], 0.0)
                    idx = ioff + g * L
                    cur = plsc.load_gather(pm_v, [idx])
                    plsc.store_scatter(pm_v, [idx], jnp.maximum(cur, h))
                    plsc.addupdate_scatter(ps_v, [idx], h)
                return ()
            lax.fori_loop(0, RC, row, ())
        return ()
    lax.fori_loop(0, (NRC + NW - 1) // NW, do_chunk, ())

    pltpu.sync_copy(pm_v, pmax_hbm.at[wid])
    pltpu.sync_copy(ps_v, psum_hbm.at[wid])


def _pool_partials(acc, y, dinv, batch, bc):
    acc0f = acc[0].reshape(N * F)
    acc1f = acc[1].reshape(N * F)
    yf = y.reshape(N * F)
    mesh = plsc.VectorSubcoreMesh(core_axis_name="c", subcore_axis_name="s")
    return pl.kernel(
        _pool_body,
        out_type=(jax.ShapeDtypeStruct((NW, NG * F), jnp.float32),
                  jax.ShapeDtypeStruct((NW, NG * F), jnp.float32)),
        mesh=mesh,
        compiler_params=pltpu.CompilerParams(needs_layout_passes=False),
        scratch_types=[
            pltpu.VMEM((RC * F,), jnp.float32),
            pltpu.VMEM((RC * F,), jnp.float32),
            pltpu.VMEM((RC * F,), jnp.float32),
            pltpu.VMEM((RC + L,), jnp.float32),
            pltpu.VMEM((RC + L,), jnp.int32),
            pltpu.VMEM((F,), jnp.float32),
            pltpu.VMEM((NG * F,), jnp.float32),
            pltpu.VMEM((NG * F,), jnp.float32),
        ],
    )(acc0f, acc1f, yf, dinv, batch, bc)


def kernel(x, edge_index, batch, Wc, bc, W1, b1, W2, b2, W3, b3):
    src = edge_index[0]
    dst = edge_index[1]

    degp = _deg_partials(dst)
    deg = 1.0 + degp[0] + degp[1]
    dinv = lax.rsqrt(deg)

    xw = x @ Wc
    y = xw * dinv[:, None]
    accp = _agg_partials(y, src, dst)
    pmaxp, psump = _pool_partials(accp, y, dinv, batch, bc)

    gmax = jnp.max(pmaxp.reshape(NW, NG, F), axis=0)
    s = jnp.sum(psump.reshape(NW, NG, F), axis=0)
    cnt = jnp.sum((batch[:, None] == jnp.arange(NG)[None, :]).astype(jnp.float32),
                  axis=0)
    gmean = s / jnp.maximum(cnt, 1.0)[:, None]
    out = jax.nn.relu(jnp.concatenate([gmax, gmean], axis=1))
    out = jax.nn.relu(out @ W1 + b1)
    out = jax.nn.relu(out @ W2 + b2)
    out = out @ W3 + b3
    return jax.nn.log_softmax(out, axis=-1)
